# baseline (speedup 1.0000x reference)
import functools

import jax
import jax.numpy as jnp
from jax import lax
from jax.experimental import pallas as pl
from jax.experimental.pallas import tpu as pltpu
from jax.experimental.pallas import tpu_sc as plsc

N = 10000
E = 320000
D = 128
DH = D // 2
NPAD = 10240
NC = 2
NS = 16
NW = NC * NS
B = 80
NBD = (E // NW) // B
NB2 = (E // NS) // B
NBUF = 2
NBP2 = 256
RPT = NPAD // NS

_mesh = plsc.VectorSubcoreMesh(core_axis_name="c", subcore_axis_name="s")


@functools.partial(
    pl.kernel,
    out_type=jax.ShapeDtypeStruct((NC, NPAD), jnp.float32),
    mesh=_mesh,
    scratch_types=[
        pltpu.VMEM((NBD, B), jnp.int32),
        pltpu.VMEM((B,), jnp.float32),
        pltpu.VMEM((RPT,), jnp.float32),
        pltpu.VMEM_SHARED((NPAD,), jnp.float32),
    ],
)
def _sc_deg(dst_hbm, out_hbm, dstv, onesv, zerov, acc):
    cid = lax.axis_index("c")
    sid = lax.axis_index("s")
    wid = sid * NC + cid
    for i in range(B // 16):
        onesv[pl.ds(i * 16, 16)] = jnp.ones((16,), jnp.float32)
    for i in range(RPT // 16):
        zerov[pl.ds(i * 16, 16)] = jnp.zeros((16,), jnp.float32)
    sl = pl.ds(sid * RPT, RPT)
    pltpu.sync_copy(zerov, acc.at[sl])
    pltpu.sync_copy(dst_hbm.at[wid], dstv)
    plsc.subcore_barrier()

    def body(j, carry):
        pltpu.sync_copy(onesv, acc.at[dstv.at[j]], add=True)
        return carry

    lax.fori_loop(0, NBD, body, 0)
    plsc.subcore_barrier()
    pltpu.sync_copy(acc.at[sl], out_hbm.at[cid, sl])


@functools.partial(
    pl.kernel,
    out_type=jax.ShapeDtypeStruct((NC, NPAD, DH), jnp.float32),
    mesh=_mesh,
    compiler_params=pltpu.CompilerParams(use_tc_tiling_on_sc=False),
    scratch_types=[
        pltpu.VMEM((NBP2, B), jnp.int32),
        pltpu.VMEM((NB2, B), jnp.int32),
        [pltpu.VMEM((B, DH), jnp.float32)] * NBUF,
        pltpu.VMEM_SHARED((N, DH), jnp.float32),
        pltpu.VMEM_SHARED((N, DH), jnp.float32),
        [pltpu.SemaphoreType.DMA] * NBUF,
        [pltpu.SemaphoreType.DMA] * NBUF,
    ],
)
def _sc_spmm(u_hbm, src_hbm, dst_hbm, out_hbm, srcv, dstv, rows, acc,
             ustage, gsem, ssem):
    cid = lax.axis_index("c")
    sid = lax.axis_index("s")
    rpt = N // NS
    sl = pl.ds(sid * rpt, rpt)
    pltpu.sync_copy(u_hbm.at[cid, sl], acc.at[sl])
    pltpu.sync_copy(u_hbm.at[cid, sl], ustage.at[sl])
    pltpu.sync_copy(src_hbm.at[sid], srcv)
    pltpu.sync_copy(dst_hbm.at[sid], dstv)

    def g_start(jj, k):
        pltpu.async_copy(ustage.at[srcv.at[jj]], rows[k], gsem[k])

    def g_wait(jj, k):
        pltpu.make_async_copy(ustage.at[srcv.at[jj]], rows[k], gsem[k]).wait()

    def s_start(jj, k):
        pltpu.async_copy(rows[k], acc.at[dstv.at[jj]], ssem[k], add=True)

    def s_wait(jj, k):
        pltpu.make_async_copy(rows[k], acc.at[dstv.at[jj]], ssem[k]).wait()

    plsc.subcore_barrier()
    for k in range(NBUF):
        g_start(k, k)

    def body(i, carry):
        j = i * NBUF
        for k in range(NBUF):
            g_wait(j + k, k)
            s_start(j + k, k)
        for k in range(NBUF):
            s_wait(j + k, k)
            g_start(j + NBUF + k, k)
        return carry

    lax.fori_loop(0, NB2 // NBUF, body, 0)
    for k in range(NBUF):
        g_wait(NB2 + k, k)
    plsc.subcore_barrier()
    pltpu.sync_copy(acc.at[sl], out_hbm.at[cid, sl])


_R = 512
_G = NPAD // _R


def _tc1_body(x_ref, w_ref, degb_ref, u_ref, dinv_ref):
    h = jnp.dot(x_ref[...], w_ref[0], preferred_element_type=jnp.float32)
    deg = degb_ref[0] + degb_ref[1] + 1.0
    dinv = lax.rsqrt(deg)
    dinv_ref[0] = dinv
    u_ref[0] = h * dinv


def _tc1(x_pad, W1s, degb):
    return pl.pallas_call(
        _tc1_body,
        grid=(_G, NC),
        in_specs=[
            pl.BlockSpec((_R, D), lambda i, c: (i, 0)),
            pl.BlockSpec((1, D, DH), lambda i, c: (c, 0, 0)),
            pl.BlockSpec((NC, _R, DH), lambda i, c: (0, i, 0)),
        ],
        out_specs=[
            pl.BlockSpec((1, _R, DH), lambda i, c: (c, i, 0)),
            pl.BlockSpec((1, _R, DH), lambda i, c: (c, i, 0)),
        ],
        out_shape=[
            jax.ShapeDtypeStruct((NC, NPAD, DH), jnp.float32),
            jax.ShapeDtypeStruct((NC, NPAD, DH), jnp.float32),
        ],
    )(x_pad, W1s, degb)


def _tc2_body(s_ref, dv_ref, b1_ref, w2_ref, u2_ref):
    s = jnp.concatenate([s_ref[0], s_ref[1]], axis=1)
    dv = jnp.concatenate([dv_ref[0], dv_ref[0]], axis=1)
    z = jnp.maximum(s * dv + b1_ref[...], 0.0)
    h2 = jnp.dot(z, w2_ref[0], preferred_element_type=jnp.float32)
    u2_ref[0] = h2 * dv_ref[0]


def _tc2(S1, dinvc, b1r, W2s):
    return pl.pallas_call(
        _tc2_body,
        grid=(_G, NC),
        in_specs=[
            pl.BlockSpec((NC, _R, DH), lambda i, c: (0, i, 0)),
            pl.BlockSpec((NC, _R, DH), lambda i, c: (0, i, 0)),
            pl.BlockSpec((1, D), lambda i, c: (0, 0)),
            pl.BlockSpec((1, D, DH), lambda i, c: (c, 0, 0)),
        ],
        out_specs=pl.BlockSpec((1, _R, DH), lambda i, c: (c, i, 0)),
        out_shape=jax.ShapeDtypeStruct((NC, NPAD, DH), jnp.float32),
    )(S1, dinvc, b1r, W2s)


def _tc3_body(s_ref, dv_ref, b2_ref, o_ref):
    s = jnp.concatenate([s_ref[0], s_ref[1]], axis=1)
    dv = jnp.concatenate([dv_ref[0], dv_ref[0]], axis=1)
    o_ref[...] = s * dv + b2_ref[...]


def _tc3(S2, dinvc, b2r):
    return pl.pallas_call(
        _tc3_body,
        grid=(_G,),
        in_specs=[
            pl.BlockSpec((NC, _R, DH), lambda i: (0, i, 0)),
            pl.BlockSpec((NC, _R, DH), lambda i: (0, i, 0)),
            pl.BlockSpec((1, D), lambda i: (0, 0)),
        ],
        out_specs=pl.BlockSpec((_R, D), lambda i: (i, 0)),
        out_shape=jax.ShapeDtypeStruct((NPAD, D), jnp.float32),
    )(S2, dinvc, b2r)


def kernel(x, edge_index, W1, b1, W2, b2):
    x_pad = jnp.concatenate(
        [x, jnp.zeros((NPAD - N, D), jnp.float32)], axis=0)
    dst_deg = edge_index[1].reshape(NW, NBD, B)
    src16 = jnp.concatenate(
        [edge_index[0].reshape(NS, NB2, B),
         jnp.zeros((NS, NBP2 - NB2, B), jnp.int32)], axis=1)
    dst16 = edge_index[1].reshape(NS, NB2, B)
    b1r = b1.reshape(1, D)
    b2r = b2.reshape(1, D)
    W1s = W1.reshape(D, NC, DH).transpose(1, 0, 2)
    W2s = W2.reshape(D, NC, DH).transpose(1, 0, 2)

    degp = _sc_deg(dst_deg)
    degb = jnp.broadcast_to(degp[:, :, None], (NC, NPAD, DH))
    U1c, dinvc = _tc1(x_pad, W1s, degb)
    S1 = _sc_spmm(U1c, src16, dst16)
    U2c = _tc2(S1, dinvc, b1r, W2s)
    S2 = _sc_spmm(U2c, src16, dst16)
    out = _tc3(S2, dinvc, b2r)
    return out[:N]


# speedup vs baseline: 1.1245x; 1.1245x over previous
import functools

import jax
import jax.numpy as jnp
from jax import lax
from jax.experimental import pallas as pl
from jax.experimental.pallas import tpu as pltpu
from jax.experimental.pallas import tpu_sc as plsc

N = 10000
E = 320000
D = 128
DH = D // 2
NPAD = 10240
NC = 2
NS = 16
NW = NC * NS
B = 80
NBD = (E // NW) // B
NB2 = (E // NS) // B
NBUF = 2
NBP2 = 256
RPT = NPAD // NS

_mesh = plsc.VectorSubcoreMesh(core_axis_name="c", subcore_axis_name="s")


@functools.partial(
    pl.kernel,
    out_type=jax.ShapeDtypeStruct((NC, NPAD), jnp.float32),
    mesh=_mesh,
    scratch_types=[
        pltpu.VMEM((NBD, B), jnp.int32),
        pltpu.VMEM((B,), jnp.float32),
        pltpu.VMEM((RPT,), jnp.float32),
        pltpu.VMEM_SHARED((NPAD,), jnp.float32),
    ],
)
def _sc_deg(dst_hbm, out_hbm, dstv, onesv, zerov, acc):
    cid = lax.axis_index("c")
    sid = lax.axis_index("s")
    wid = sid * NC + cid
    for i in range(B // 16):
        onesv[pl.ds(i * 16, 16)] = jnp.ones((16,), jnp.float32)
    for i in range(RPT // 16):
        zerov[pl.ds(i * 16, 16)] = jnp.zeros((16,), jnp.float32)
    sl = pl.ds(sid * RPT, RPT)
    pltpu.sync_copy(zerov, acc.at[sl])
    pltpu.sync_copy(dst_hbm.at[wid], dstv)
    plsc.subcore_barrier()

    def body(j, carry):
        pltpu.sync_copy(onesv, acc.at[dstv.at[j]], add=True)
        return carry

    lax.fori_loop(0, NBD, body, 0)
    plsc.subcore_barrier()
    pltpu.sync_copy(acc.at[sl], out_hbm.at[cid, sl])


@functools.partial(
    pl.kernel,
    out_type=jax.ShapeDtypeStruct((NC * N, DH), jnp.float32),
    mesh=_mesh,
    compiler_params=pltpu.CompilerParams(use_tc_tiling_on_sc=False),
    scratch_types=[
        pltpu.VMEM((NBP2, B), jnp.int32),
        pltpu.VMEM((NB2, B), jnp.int32),
        [pltpu.VMEM((B, DH), jnp.float32)] * NBUF,
        pltpu.VMEM_SHARED((N, DH), jnp.float32),
        pltpu.VMEM_SHARED((N, DH), jnp.float32),
        [pltpu.SemaphoreType.DMA] * NBUF,
        [pltpu.SemaphoreType.DMA] * NBUF,
    ],
)
def _sc_spmm(u_hbm, src_hbm, dst_hbm, out_hbm, srcv, dstv, rows, acc,
             ustage, gsem, ssem):
    cid = lax.axis_index("c")
    sid = lax.axis_index("s")
    rpt = N // NS
    hsl = pl.ds(cid * N + sid * rpt, rpt)
    sl = pl.ds(sid * rpt, rpt)
    pltpu.sync_copy(u_hbm.at[hsl], acc.at[sl])
    pltpu.sync_copy(u_hbm.at[hsl], ustage.at[sl])
    pltpu.sync_copy(src_hbm.at[sid], srcv)
    pltpu.sync_copy(dst_hbm.at[sid], dstv)

    def g_start(jj, k):
        pltpu.async_copy(ustage.at[srcv.at[jj]], rows[k], gsem[k])

    def g_wait(jj, k):
        pltpu.make_async_copy(ustage.at[srcv.at[jj]], rows[k], gsem[k]).wait()

    def s_start(jj, k):
        pltpu.async_copy(rows[k], acc.at[dstv.at[jj]], ssem[k], add=True)

    def s_wait(jj, k):
        pltpu.make_async_copy(rows[k], acc.at[dstv.at[jj]], ssem[k]).wait()

    plsc.subcore_barrier()
    for k in range(NBUF):
        g_start(k, k)

    def body(i, carry):
        j = i * NBUF
        for k in range(NBUF):
            g_wait(j + k, k)
            s_start(j + k, k)
        for k in range(NBUF):
            s_wait(j + k, k)
            g_start(j + NBUF + k, k)
        return carry

    lax.fori_loop(0, NB2 // NBUF, body, 0)
    for k in range(NBUF):
        g_wait(NB2 + k, k)
    plsc.subcore_barrier()
    pltpu.sync_copy(acc.at[sl], out_hbm.at[hsl])


_R = 400
_G = N // _R


def _tc1_body(x_ref, w_ref, degb_ref, u_ref):
    h = jnp.dot(x_ref[...], w_ref[0], preferred_element_type=jnp.float32)
    dinv = lax.rsqrt(degb_ref[0] + degb_ref[1] + 1.0)
    u_ref[...] = h * dinv


def _tc1(x, W1s, degb):
    return pl.pallas_call(
        _tc1_body,
        grid=(_G, NC),
        in_specs=[
            pl.BlockSpec((_R, D), lambda i, c: (i, 0)),
            pl.BlockSpec((1, D, DH), lambda i, c: (c, 0, 0)),
            pl.BlockSpec((NC, _R, DH), lambda i, c: (0, i, 0)),
        ],
        out_specs=pl.BlockSpec((_R, DH), lambda i, c: (c * _G + i, 0)),
        out_shape=jax.ShapeDtypeStruct((NC * N, DH), jnp.float32),
    )(x, W1s, degb)


def _tc2_body(s0_ref, s1_ref, degb_ref, b1_ref, w2_ref, u2_ref):
    s = jnp.concatenate([s0_ref[...], s1_ref[...]], axis=1)
    dinv = lax.rsqrt(degb_ref[0] + degb_ref[1] + 1.0)
    dv = jnp.concatenate([dinv, dinv], axis=1)
    z = jnp.maximum(s * dv + b1_ref[...], 0.0)
    h2 = jnp.dot(z, w2_ref[0], preferred_element_type=jnp.float32)
    u2_ref[...] = h2 * dinv


def _tc2(S1, degb, b1r, W2s):
    return pl.pallas_call(
        _tc2_body,
        grid=(_G, NC),
        in_specs=[
            pl.BlockSpec((_R, DH), lambda i, c: (i, 0)),
            pl.BlockSpec((_R, DH), lambda i, c: (_G + i, 0)),
            pl.BlockSpec((NC, _R, DH), lambda i, c: (0, i, 0)),
            pl.BlockSpec((1, D), lambda i, c: (0, 0)),
            pl.BlockSpec((1, D, DH), lambda i, c: (c, 0, 0)),
        ],
        out_specs=pl.BlockSpec((_R, DH), lambda i, c: (c * _G + i, 0)),
        out_shape=jax.ShapeDtypeStruct((NC * N, DH), jnp.float32),
    )(S1, S1, degb, b1r, W2s)


def _tc3_body(s0_ref, s1_ref, degb_ref, b2_ref, o_ref):
    s = jnp.concatenate([s0_ref[...], s1_ref[...]], axis=1)
    dinv = lax.rsqrt(degb_ref[0] + degb_ref[1] + 1.0)
    dv = jnp.concatenate([dinv, dinv], axis=1)
    o_ref[...] = s * dv + b2_ref[...]


def _tc3(S2, degb, b2r):
    return pl.pallas_call(
        _tc3_body,
        grid=(_G,),
        in_specs=[
            pl.BlockSpec((_R, DH), lambda i: (i, 0)),
            pl.BlockSpec((_R, DH), lambda i: (_G + i, 0)),
            pl.BlockSpec((NC, _R, DH), lambda i: (0, i, 0)),
            pl.BlockSpec((1, D), lambda i: (0, 0)),
        ],
        out_specs=pl.BlockSpec((_R, D), lambda i: (i, 0)),
        out_shape=jax.ShapeDtypeStruct((N, D), jnp.float32),
    )(S2, S2, degb, b2r)


def kernel(x, edge_index, W1, b1, W2, b2):
    dst_deg = edge_index[1].reshape(NW, NBD, B)
    src16 = jnp.concatenate(
        [edge_index[0].reshape(NS, NB2, B),
         jnp.zeros((NS, NBP2 - NB2, B), jnp.int32)], axis=1)
    dst16 = edge_index[1].reshape(NS, NB2, B)
    b1r = b1.reshape(1, D)
    b2r = b2.reshape(1, D)
    W1s = W1.reshape(D, NC, DH).transpose(1, 0, 2)
    W2s = W2.reshape(D, NC, DH).transpose(1, 0, 2)

    degp = _sc_deg(dst_deg)
    degb = jnp.broadcast_to(degp[:, :N, None], (NC, N, DH))
    U1 = _tc1(x, W1s, degb)
    S1 = _sc_spmm(U1, src16, dst16)
    U2 = _tc2(S1, degb, b1r, W2s)
    S2 = _sc_spmm(U2, src16, dst16)
    return _tc3(S2, degb, b2r)


# speedup vs baseline: 1.3310x; 1.1836x over previous
import functools

import jax
import jax.numpy as jnp
from jax import lax
from jax.experimental import pallas as pl
from jax.experimental.pallas import tpu as pltpu
from jax.experimental.pallas import tpu_sc as plsc

N = 10000
E = 320000
D = 128
DH = D // 2
NPAD = 10240
NC = 2
NS = 16
NW = NC * NS
B = 80
NBD = (E // NW) // B
NB2 = (E // NS) // B
NBUF = 2
NBP2 = 256
RPT = NPAD // NS

_mesh = plsc.VectorSubcoreMesh(core_axis_name="c", subcore_axis_name="s")


@functools.partial(
    pl.kernel,
    out_type=jax.ShapeDtypeStruct((NC, NPAD), jnp.float32),
    mesh=_mesh,
    scratch_types=[
        pltpu.VMEM((NBD, B), jnp.int32),
        pltpu.VMEM((B,), jnp.float32),
        pltpu.VMEM((RPT,), jnp.float32),
        pltpu.VMEM_SHARED((NPAD,), jnp.float32),
    ],
)
def _sc_deg(dst_hbm, out_hbm, dstv, onesv, zerov, acc):
    cid = lax.axis_index("c")
    sid = lax.axis_index("s")
    wid = sid * NC + cid
    for i in range(B // 16):
        onesv[pl.ds(i * 16, 16)] = jnp.ones((16,), jnp.float32)
    for i in range(RPT // 16):
        zerov[pl.ds(i * 16, 16)] = jnp.zeros((16,), jnp.float32)
    sl = pl.ds(sid * RPT, RPT)
    pltpu.sync_copy(zerov, acc.at[sl])
    pltpu.sync_copy(dst_hbm.at[wid], dstv)
    plsc.subcore_barrier()

    def body(j, carry):
        pltpu.sync_copy(onesv, acc.at[dstv.at[j]], add=True)
        return carry

    lax.fori_loop(0, NBD, body, 0)
    plsc.subcore_barrier()
    pltpu.sync_copy(acc.at[sl], out_hbm.at[cid, sl])


@functools.partial(
    pl.kernel,
    out_type=jax.ShapeDtypeStruct((N, D), jnp.float32),
    mesh=_mesh,
    compiler_params=pltpu.CompilerParams(use_tc_tiling_on_sc=False),
    scratch_types=[
        pltpu.VMEM((NBP2, B), jnp.int32),
        pltpu.VMEM((NB2, B), jnp.int32),
        [pltpu.VMEM((B, DH), jnp.float32)] * NBUF,
        pltpu.VMEM_SHARED((N, DH), jnp.float32),
        pltpu.VMEM_SHARED((N, DH), jnp.float32),
        [pltpu.SemaphoreType.DMA] * NBUF,
        [pltpu.SemaphoreType.DMA] * NBUF,
    ],
)
def _sc_spmm(u_hbm, src_hbm, dst_hbm, out_hbm, srcv, dstv, rows, acc,
             ustage, gsem, ssem):
    cid = lax.axis_index("c")
    sid = lax.axis_index("s")
    rpt = N // NS
    sl = pl.ds(sid * rpt, rpt)
    csl = pl.ds(cid * DH, DH)
    pltpu.sync_copy(u_hbm.at[sl, csl], acc.at[sl])
    pltpu.sync_copy(u_hbm.at[sl, csl], ustage.at[sl])
    pltpu.sync_copy(src_hbm.at[sid], srcv)
    pltpu.sync_copy(dst_hbm.at[sid], dstv)

    def g_start(jj, k):
        pltpu.async_copy(ustage.at[srcv.at[jj]], rows[k], gsem[k])

    def g_wait(jj, k):
        pltpu.make_async_copy(ustage.at[srcv.at[jj]], rows[k], gsem[k]).wait()

    def s_start(jj, k):
        pltpu.async_copy(rows[k], acc.at[dstv.at[jj]], ssem[k], add=True)

    def s_wait(jj, k):
        pltpu.make_async_copy(rows[k], acc.at[dstv.at[jj]], ssem[k]).wait()

    plsc.subcore_barrier()
    for k in range(NBUF):
        g_start(k, k)

    def body(i, carry):
        j = i * NBUF
        for k in range(NBUF):
            g_wait(j + k, k)
            s_start(j + k, k)
        for k in range(NBUF):
            s_wait(j + k, k)
            g_start(j + NBUF + k, k)
        return carry

    lax.fori_loop(0, NB2 // NBUF, body, 0)
    for k in range(NBUF):
        g_wait(NB2 + k, k)
    plsc.subcore_barrier()
    pltpu.sync_copy(acc.at[sl], out_hbm.at[sl, csl])


_R = 400
_G = N // _R


def _dv(degb_ref):
    dinv = lax.rsqrt(degb_ref[0] + degb_ref[1] + 1.0)
    return jnp.concatenate([dinv, dinv], axis=1)


def _tc1_body(x_ref, w_ref, degb_ref, u_ref):
    h = jnp.dot(x_ref[...], w_ref[...], preferred_element_type=jnp.float32)
    u_ref[...] = h * _dv(degb_ref)


def _tc1(x, W1, degb):
    return pl.pallas_call(
        _tc1_body,
        grid=(_G,),
        in_specs=[
            pl.BlockSpec((_R, D), lambda i: (i, 0)),
            pl.BlockSpec((D, D), lambda i: (0, 0)),
            pl.BlockSpec((NC, _R, DH), lambda i: (0, i, 0)),
        ],
        out_specs=pl.BlockSpec((_R, D), lambda i: (i, 0)),
        out_shape=jax.ShapeDtypeStruct((N, D), jnp.float32),
    )(x, W1, degb)


def _tc2_body(s_ref, degb_ref, b1_ref, w2_ref, u2_ref):
    dv = _dv(degb_ref)
    z = jnp.maximum(s_ref[...] * dv + b1_ref[...], 0.0)
    h2 = jnp.dot(z, w2_ref[...], preferred_element_type=jnp.float32)
    u2_ref[...] = h2 * dv


def _tc2(S1, degb, b1r, W2):
    return pl.pallas_call(
        _tc2_body,
        grid=(_G,),
        in_specs=[
            pl.BlockSpec((_R, D), lambda i: (i, 0)),
            pl.BlockSpec((NC, _R, DH), lambda i: (0, i, 0)),
            pl.BlockSpec((1, D), lambda i: (0, 0)),
            pl.BlockSpec((D, D), lambda i: (0, 0)),
        ],
        out_specs=pl.BlockSpec((_R, D), lambda i: (i, 0)),
        out_shape=jax.ShapeDtypeStruct((N, D), jnp.float32),
    )(S1, degb, b1r, W2)


def _tc3_body(s_ref, degb_ref, b2_ref, o_ref):
    o_ref[...] = s_ref[...] * _dv(degb_ref) + b2_ref[...]


def _tc3(S2, degb, b2r):
    return pl.pallas_call(
        _tc3_body,
        grid=(_G,),
        in_specs=[
            pl.BlockSpec((_R, D), lambda i: (i, 0)),
            pl.BlockSpec((NC, _R, DH), lambda i: (0, i, 0)),
            pl.BlockSpec((1, D), lambda i: (0, 0)),
        ],
        out_specs=pl.BlockSpec((_R, D), lambda i: (i, 0)),
        out_shape=jax.ShapeDtypeStruct((N, D), jnp.float32),
    )(S2, degb, b2r)


def kernel(x, edge_index, W1, b1, W2, b2):
    dst_deg = edge_index[1].reshape(NW, NBD, B)
    src16 = jnp.concatenate(
        [edge_index[0].reshape(NS, NB2, B),
         jnp.zeros((NS, NBP2 - NB2, B), jnp.int32)], axis=1)
    dst16 = edge_index[1].reshape(NS, NB2, B)
    b1r = b1.reshape(1, D)
    b2r = b2.reshape(1, D)

    degp = _sc_deg(dst_deg)
    degb = jnp.broadcast_to(degp[:, :N, None], (NC, N, DH))
    U1 = _tc1(x, W1, degb)
    S1 = _sc_spmm(U1, src16, dst16)
    U2 = _tc2(S1, degb, b1r, W2)
    S2 = _sc_spmm(U2, src16, dst16)
    return _tc3(S2, degb, b2r)


# speedup vs baseline: 1.3318x; 1.0006x over previous
import functools

import jax
import jax.numpy as jnp
from jax import lax
from jax.experimental import pallas as pl
from jax.experimental.pallas import tpu as pltpu
from jax.experimental.pallas import tpu_sc as plsc

N = 10000
E = 320000
D = 128
DH = D // 2
NPAD = 10240
NC = 2
NS = 16
NW = NC * NS
B = 80
NBD = (E // NW) // B
NB2 = (E // NS) // B
NBUF = 2
NBP2 = 256
RPT = NPAD // NS

_mesh = plsc.VectorSubcoreMesh(core_axis_name="c", subcore_axis_name="s")


@functools.partial(
    pl.kernel,
    out_type=jax.ShapeDtypeStruct((NC, NPAD), jnp.float32),
    mesh=_mesh,
    scratch_types=[
        pltpu.VMEM((NBD, B), jnp.int32),
        pltpu.VMEM((B,), jnp.float32),
        pltpu.VMEM((RPT,), jnp.float32),
        pltpu.VMEM_SHARED((NPAD,), jnp.float32),
    ],
)
def _sc_deg(dst_hbm, out_hbm, dstv, onesv, zerov, acc):
    cid = lax.axis_index("c")
    sid = lax.axis_index("s")
    wid = sid * NC + cid
    for i in range(B // 16):
        onesv[pl.ds(i * 16, 16)] = jnp.ones((16,), jnp.float32)
    for i in range(RPT // 16):
        zerov[pl.ds(i * 16, 16)] = jnp.zeros((16,), jnp.float32)
    sl = pl.ds(sid * RPT, RPT)
    pltpu.sync_copy(zerov, acc.at[sl])
    pltpu.sync_copy(dst_hbm.at[wid], dstv)
    plsc.subcore_barrier()

    def body(j, carry):
        pltpu.sync_copy(onesv, acc.at[dstv.at[j]], add=True)
        return carry

    lax.fori_loop(0, NBD, body, 0)
    plsc.subcore_barrier()
    pltpu.sync_copy(acc.at[sl], out_hbm.at[cid, sl])


@functools.partial(
    pl.kernel,
    out_type=jax.ShapeDtypeStruct((N, D), jnp.float32),
    mesh=_mesh,
    compiler_params=pltpu.CompilerParams(use_tc_tiling_on_sc=False),
    scratch_types=[
        pltpu.VMEM((NB2, B), jnp.int32),
        pltpu.VMEM((NB2, B), jnp.int32),
        [pltpu.VMEM((B, DH), jnp.float32)] * NBUF,
        pltpu.VMEM_SHARED((N, DH), jnp.float32),
        pltpu.VMEM_SHARED((N, DH), jnp.float32),
        [pltpu.SemaphoreType.DMA] * NBUF,
        [pltpu.SemaphoreType.DMA] * NBUF,
    ],
)
def _sc_spmm(u_hbm, src_hbm, dst_hbm, out_hbm, srcv, dstv, rows, acc,
             ustage, gsem, ssem):
    cid = lax.axis_index("c")
    sid = lax.axis_index("s")
    rpt = N // NS
    sl = pl.ds(sid * rpt, rpt)
    csl = pl.ds(cid * DH, DH)
    pltpu.sync_copy(u_hbm.at[sl, csl], acc.at[sl])
    pltpu.sync_copy(u_hbm.at[sl, csl], ustage.at[sl])
    pltpu.sync_copy(src_hbm.at[sid], srcv)
    pltpu.sync_copy(dst_hbm.at[sid], dstv)

    def g_start(jj, k):
        jw = lax.rem(jj, NB2)
        pltpu.async_copy(ustage.at[srcv.at[jw]], rows[k], gsem[k])

    def g_wait(jj, k):
        jw = lax.rem(jj, NB2)
        pltpu.make_async_copy(ustage.at[srcv.at[jw]], rows[k], gsem[k]).wait()

    def s_start(jj, k):
        pltpu.async_copy(rows[k], acc.at[dstv.at[jj]], ssem[k], add=True)

    def s_wait(jj, k):
        pltpu.make_async_copy(rows[k], acc.at[dstv.at[jj]], ssem[k]).wait()

    plsc.subcore_barrier()
    for k in range(NBUF):
        g_start(k, k)

    def body(i, carry):
        j = i * NBUF
        for k in range(NBUF):
            g_wait(j + k, k)
            s_start(j + k, k)
        for k in range(NBUF):
            s_wait(j + k, k)
            g_start(j + NBUF + k, k)
        return carry

    lax.fori_loop(0, NB2 // NBUF, body, 0)
    for k in range(NBUF):
        g_wait(NB2 + k, k)
    plsc.subcore_barrier()
    pltpu.sync_copy(acc.at[sl], out_hbm.at[sl, csl])


_R = 400
_G = N // _R


def _dv(degb_ref):
    dinv = lax.rsqrt(degb_ref[0] + degb_ref[1] + 1.0)
    return jnp.concatenate([dinv, dinv], axis=1)


def _tc1_body(x_ref, w_ref, degb_ref, u_ref):
    h = jnp.dot(x_ref[...], w_ref[...], preferred_element_type=jnp.float32)
    u_ref[...] = h * _dv(degb_ref)


def _tc1(x, W1, degb):
    return pl.pallas_call(
        _tc1_body,
        grid=(_G,),
        in_specs=[
            pl.BlockSpec((_R, D), lambda i: (i, 0)),
            pl.BlockSpec((D, D), lambda i: (0, 0)),
            pl.BlockSpec((NC, _R, DH), lambda i: (0, i, 0)),
        ],
        out_specs=pl.BlockSpec((_R, D), lambda i: (i, 0)),
        out_shape=jax.ShapeDtypeStruct((N, D), jnp.float32),
    )(x, W1, degb)


def _tc2_body(s_ref, degb_ref, b1_ref, w2_ref, u2_ref):
    dv = _dv(degb_ref)
    z = jnp.maximum(s_ref[...] * dv + b1_ref[...], 0.0)
    h2 = jnp.dot(z, w2_ref[...], preferred_element_type=jnp.float32)
    u2_ref[...] = h2 * dv


def _tc2(S1, degb, b1r, W2):
    return pl.pallas_call(
        _tc2_body,
        grid=(_G,),
        in_specs=[
            pl.BlockSpec((_R, D), lambda i: (i, 0)),
            pl.BlockSpec((NC, _R, DH), lambda i: (0, i, 0)),
            pl.BlockSpec((1, D), lambda i: (0, 0)),
            pl.BlockSpec((D, D), lambda i: (0, 0)),
        ],
        out_specs=pl.BlockSpec((_R, D), lambda i: (i, 0)),
        out_shape=jax.ShapeDtypeStruct((N, D), jnp.float32),
    )(S1, degb, b1r, W2)


def _tc3_body(s_ref, degb_ref, b2_ref, o_ref):
    o_ref[...] = s_ref[...] * _dv(degb_ref) + b2_ref[...]


def _tc3(S2, degb, b2r):
    return pl.pallas_call(
        _tc3_body,
        grid=(_G,),
        in_specs=[
            pl.BlockSpec((_R, D), lambda i: (i, 0)),
            pl.BlockSpec((NC, _R, DH), lambda i: (0, i, 0)),
            pl.BlockSpec((1, D), lambda i: (0, 0)),
        ],
        out_specs=pl.BlockSpec((_R, D), lambda i: (i, 0)),
        out_shape=jax.ShapeDtypeStruct((N, D), jnp.float32),
    )(S2, degb, b2r)


def kernel(x, edge_index, W1, b1, W2, b2):
    dst_deg = edge_index[1].reshape(NW, NBD, B)
    src16 = edge_index[0].reshape(NS, NB2, B)
    dst16 = edge_index[1].reshape(NS, NB2, B)
    b1r = b1.reshape(1, D)
    b2r = b2.reshape(1, D)

    degp = _sc_deg(dst_deg)
    degb = jnp.broadcast_to(degp[:, :N, None], (NC, N, DH))
    U1 = _tc1(x, W1, degb)
    S1 = _sc_spmm(U1, src16, dst16)
    U2 = _tc2(S1, degb, b1r, W2)
    S2 = _sc_spmm(U2, src16, dst16)
    return _tc3(S2, degb, b2r)


# speedup vs baseline: 1.3650x; 1.0249x over previous
import functools

import jax
import jax.numpy as jnp
from jax import lax
from jax.experimental import pallas as pl
from jax.experimental.pallas import tpu as pltpu
from jax.experimental.pallas import tpu_sc as plsc

N = 10000
E = 320000
D = 128
DH = D // 2
NPAD = 10240
NC = 2
NS = 16
NW = NC * NS
B = 80
NBD = (E // NW) // B
NB2 = (E // NS) // B
NBUF = 2
NBP2 = 256
RPT = NPAD // NS

_mesh = plsc.VectorSubcoreMesh(core_axis_name="c", subcore_axis_name="s")


@functools.partial(
    pl.kernel,
    out_type=jax.ShapeDtypeStruct((NC, NPAD), jnp.float32),
    mesh=_mesh,
    scratch_types=[
        pltpu.VMEM((NB2, B), jnp.int32),
        pltpu.VMEM((B,), jnp.float32),
        pltpu.VMEM((RPT,), jnp.float32),
        pltpu.VMEM_SHARED((NPAD,), jnp.float32),
    ],
)
def _sc_deg(edge_hbm, out_hbm, dstv, onesv, zerov, acc):
    cid = lax.axis_index("c")
    sid = lax.axis_index("s")
    for i in range(B // 16):
        onesv[pl.ds(i * 16, 16)] = jnp.ones((16,), jnp.float32)
    for i in range(RPT // 16):
        zerov[pl.ds(i * 16, 16)] = jnp.zeros((16,), jnp.float32)
    sl = pl.ds(sid * RPT, RPT)
    pltpu.sync_copy(zerov, acc.at[sl])
    pltpu.sync_copy(edge_hbm.at[1, sid], dstv)
    plsc.subcore_barrier()

    def body(j, carry):
        pltpu.sync_copy(onesv, acc.at[dstv.at[j]], add=True)
        return carry

    lax.fori_loop(cid * NBD, cid * NBD + NBD, body, 0)
    plsc.subcore_barrier()
    pltpu.sync_copy(acc.at[sl], out_hbm.at[cid, sl])


@functools.partial(
    pl.kernel,
    out_type=jax.ShapeDtypeStruct((N, D), jnp.float32),
    mesh=_mesh,
    compiler_params=pltpu.CompilerParams(use_tc_tiling_on_sc=False),
    scratch_types=[
        pltpu.VMEM((NB2, B), jnp.int32),
        pltpu.VMEM((NB2, B), jnp.int32),
        [pltpu.VMEM((B, DH), jnp.float32)] * NBUF,
        pltpu.VMEM_SHARED((N, DH), jnp.float32),
        pltpu.VMEM_SHARED((N, DH), jnp.float32),
        [pltpu.SemaphoreType.DMA] * NBUF,
        [pltpu.SemaphoreType.DMA] * NBUF,
    ],
)
def _sc_spmm(u_hbm, edge_hbm, out_hbm, srcv, dstv, rows, acc,
             ustage, gsem, ssem):
    cid = lax.axis_index("c")
    sid = lax.axis_index("s")
    rpt = N // NS
    sl = pl.ds(sid * rpt, rpt)
    csl = pl.ds(cid * DH, DH)
    pltpu.sync_copy(u_hbm.at[sl, csl], acc.at[sl])
    pltpu.sync_copy(u_hbm.at[sl, csl], ustage.at[sl])
    pltpu.sync_copy(edge_hbm.at[0, sid], srcv)
    pltpu.sync_copy(edge_hbm.at[1, sid], dstv)

    def g_start(jj, k):
        jw = lax.rem(jj, NB2)
        pltpu.async_copy(ustage.at[srcv.at[jw]], rows[k], gsem[k])

    def g_wait(jj, k):
        jw = lax.rem(jj, NB2)
        pltpu.make_async_copy(ustage.at[srcv.at[jw]], rows[k], gsem[k]).wait()

    def s_start(jj, k):
        pltpu.async_copy(rows[k], acc.at[dstv.at[jj]], ssem[k], add=True)

    def s_wait(jj, k):
        pltpu.make_async_copy(rows[k], acc.at[dstv.at[jj]], ssem[k]).wait()

    plsc.subcore_barrier()
    for k in range(NBUF):
        g_start(k, k)

    def body(i, carry):
        j = i * NBUF
        for k in range(NBUF):
            g_wait(j + k, k)
            s_start(j + k, k)
        for k in range(NBUF):
            s_wait(j + k, k)
            g_start(j + NBUF + k, k)
        return carry

    lax.fori_loop(0, NB2 // NBUF, body, 0)
    for k in range(NBUF):
        g_wait(NB2 + k, k)
    plsc.subcore_barrier()
    pltpu.sync_copy(acc.at[sl], out_hbm.at[sl, csl])


_R = 400
_G = N // _R


def _dv(degb_ref):
    dinv = lax.rsqrt(degb_ref[0] + degb_ref[1] + 1.0)
    return jnp.concatenate([dinv, dinv], axis=1)


def _tc1_body(x_ref, w_ref, degb_ref, u_ref):
    h = jnp.dot(x_ref[...], w_ref[...], preferred_element_type=jnp.float32)
    u_ref[...] = h * _dv(degb_ref)


def _tc1(x, W1, degb):
    return pl.pallas_call(
        _tc1_body,
        grid=(_G,),
        in_specs=[
            pl.BlockSpec((_R, D), lambda i: (i, 0)),
            pl.BlockSpec((D, D), lambda i: (0, 0)),
            pl.BlockSpec((NC, _R, DH), lambda i: (0, i, 0)),
        ],
        out_specs=pl.BlockSpec((_R, D), lambda i: (i, 0)),
        out_shape=jax.ShapeDtypeStruct((N, D), jnp.float32),
    )(x, W1, degb)


def _tc2_body(s_ref, degb_ref, b1_ref, w2_ref, u2_ref):
    dv = _dv(degb_ref)
    z = jnp.maximum(s_ref[...] * dv + b1_ref[...], 0.0)
    h2 = jnp.dot(z, w2_ref[...], preferred_element_type=jnp.float32)
    u2_ref[...] = h2 * dv


def _tc2(S1, degb, b1r, W2):
    return pl.pallas_call(
        _tc2_body,
        grid=(_G,),
        in_specs=[
            pl.BlockSpec((_R, D), lambda i: (i, 0)),
            pl.BlockSpec((NC, _R, DH), lambda i: (0, i, 0)),
            pl.BlockSpec((1, D), lambda i: (0, 0)),
            pl.BlockSpec((D, D), lambda i: (0, 0)),
        ],
        out_specs=pl.BlockSpec((_R, D), lambda i: (i, 0)),
        out_shape=jax.ShapeDtypeStruct((N, D), jnp.float32),
    )(S1, degb, b1r, W2)


def _tc3_body(s_ref, degb_ref, b2_ref, o_ref):
    o_ref[...] = s_ref[...] * _dv(degb_ref) + b2_ref[...]


def _tc3(S2, degb, b2r):
    return pl.pallas_call(
        _tc3_body,
        grid=(_G,),
        in_specs=[
            pl.BlockSpec((_R, D), lambda i: (i, 0)),
            pl.BlockSpec((NC, _R, DH), lambda i: (0, i, 0)),
            pl.BlockSpec((1, D), lambda i: (0, 0)),
        ],
        out_specs=pl.BlockSpec((_R, D), lambda i: (i, 0)),
        out_shape=jax.ShapeDtypeStruct((N, D), jnp.float32),
    )(S2, degb, b2r)


def kernel(x, edge_index, W1, b1, W2, b2):
    er = edge_index.reshape(2, NS, NB2, B)
    b1r = b1.reshape(1, D)
    b2r = b2.reshape(1, D)

    degp = _sc_deg(er)
    degb = jnp.broadcast_to(degp[:, :N, None], (NC, N, DH))
    U1 = _tc1(x, W1, degb)
    S1 = _sc_spmm(U1, er)
    U2 = _tc2(S1, degb, b1r, W2)
    S2 = _sc_spmm(U2, er)
    return _tc3(S2, degb, b2r)
